# baseline (speedup 1.0000x reference)
import functools

import jax
import jax.numpy as jnp
from jax.experimental import pallas as pl
from jax.experimental.pallas import tpu as pltpu

N = 10000
D_IN = 256
D_OUT = 256

BM = 200


def _xw_kernel(x_ref, w_ref, h_ref):
    h_ref[...] = jnp.dot(x_ref[...], w_ref[...],
                         preferred_element_type=jnp.float32)


def _spmm_kernel(adj_ref, h_ref, out_ref):
    acc = jnp.dot(adj_ref[...], h_ref[...],
                  preferred_element_type=jnp.float32)
    out_ref[...] = jnp.maximum(acc, 0.0)


@jax.jit
def kernel(x, adj, W):
    hidden = pl.pallas_call(
        _xw_kernel,
        grid=(5,),
        in_specs=[
            pl.BlockSpec((N // 5, D_IN), lambda i: (i, 0)),
            pl.BlockSpec((D_IN, D_OUT), lambda i: (0, 0)),
        ],
        out_specs=pl.BlockSpec((N // 5, D_OUT), lambda i: (i, 0)),
        out_shape=jax.ShapeDtypeStruct((N, D_OUT), jnp.float32),
        compiler_params=pltpu.CompilerParams(
            dimension_semantics=("parallel",)),
    )(x, W)

    out = pl.pallas_call(
        _spmm_kernel,
        grid=(N // BM,),
        in_specs=[
            pl.BlockSpec((BM, N), lambda i: (i, 0)),
            pl.BlockSpec((N, D_OUT), lambda i: (0, 0)),
        ],
        out_specs=pl.BlockSpec((BM, D_OUT), lambda i: (i, 0)),
        out_shape=jax.ShapeDtypeStruct((N, D_OUT), jnp.float32),
        compiler_params=pltpu.CompilerParams(
            dimension_semantics=("parallel",)),
    )(adj, hidden)

    return (out, adj)


# speedup vs baseline: 1.0021x; 1.0021x over previous
import jax
import jax.numpy as jnp
from jax.experimental import pallas as pl
from jax.experimental.pallas import tpu as pltpu

N = 10000
D_IN = 256
D_OUT = 256

BM = 200
S = 2


def _xw_kernel(x_ref, w_ref, h_ref):
    h_ref[...] = jnp.dot(x_ref[...], w_ref[...],
                         preferred_element_type=jnp.float32)


def _spmm_kernel(*refs):
    adj_refs = refs[:S]
    h_ref = refs[S]
    out_ref = refs[S + 1]
    for j in range(S):
        acc = jnp.dot(adj_refs[j][...], h_ref[...],
                      preferred_element_type=jnp.float32)
        out_ref[j * BM:(j + 1) * BM, :] = jnp.maximum(acc, 0.0)


def _adj_spec(j):
    return pl.BlockSpec((BM, N), lambda i, j=j: (S * i + j, 0))


@jax.jit
def kernel(x, adj, W):
    hidden = pl.pallas_call(
        _xw_kernel,
        grid=(5,),
        in_specs=[
            pl.BlockSpec((N // 5, D_IN), lambda i: (i, 0)),
            pl.BlockSpec((D_IN, D_OUT), lambda i: (0, 0)),
        ],
        out_specs=pl.BlockSpec((N // 5, D_OUT), lambda i: (i, 0)),
        out_shape=jax.ShapeDtypeStruct((N, D_OUT), jnp.float32),
        compiler_params=pltpu.CompilerParams(
            dimension_semantics=("parallel",)),
    )(x, W)

    out = pl.pallas_call(
        _spmm_kernel,
        grid=(N // (S * BM),),
        in_specs=[_adj_spec(j) for j in range(S)] + [
            pl.BlockSpec((N, D_OUT), lambda i: (0, 0)),
        ],
        out_specs=pl.BlockSpec((S * BM, D_OUT), lambda i: (i, 0)),
        out_shape=jax.ShapeDtypeStruct((N, D_OUT), jnp.float32),
        compiler_params=pltpu.CompilerParams(
            dimension_semantics=("parallel",)),
    )(*([adj] * S), hidden)

    return (out, adj)


# speedup vs baseline: 1.0230x; 1.0208x over previous
import jax
import jax.numpy as jnp
from jax.experimental import pallas as pl
from jax.experimental.pallas import tpu as pltpu

N = 10000
D_IN = 256
D_OUT = 256

BM = 200


def _fused_kernel(x_ref, w_ref, adj_ref, out_ref, h_scratch):
    @pl.when(pl.program_id(0) == 0)
    def _compute_hidden():
        h_scratch[...] = jnp.dot(x_ref[...], w_ref[...],
                                 preferred_element_type=jnp.float32)

    acc = jnp.dot(adj_ref[...], h_scratch[...],
                  preferred_element_type=jnp.float32)
    out_ref[...] = jnp.maximum(acc, 0.0)


@jax.jit
def kernel(x, adj, W):
    out = pl.pallas_call(
        _fused_kernel,
        grid=(N // BM,),
        in_specs=[
            pl.BlockSpec((N, D_IN), lambda i: (0, 0)),
            pl.BlockSpec((D_IN, D_OUT), lambda i: (0, 0)),
            pl.BlockSpec((BM, N), lambda i: (i, 0)),
        ],
        out_specs=pl.BlockSpec((BM, D_OUT), lambda i: (i, 0)),
        out_shape=jax.ShapeDtypeStruct((N, D_OUT), jnp.float32),
        scratch_shapes=[pltpu.VMEM((N, D_OUT), jnp.float32)],
    )(x, W, adj)

    return (out, adj)


# speedup vs baseline: 1.0274x; 1.0043x over previous
import jax
import jax.numpy as jnp
from jax.experimental import pallas as pl
from jax.experimental.pallas import tpu as pltpu

N = 10000
D_IN = 256
D_OUT = 256

BM = 400


def _fused_kernel(x_ref, w_ref, adj_ref, out_ref, h_scratch):
    @pl.when(pl.program_id(0) == 0)
    def _compute_hidden():
        h_scratch[...] = jnp.dot(x_ref[...], w_ref[...],
                                 preferred_element_type=jnp.float32)

    acc = jnp.dot(adj_ref[...], h_scratch[...],
                  preferred_element_type=jnp.float32)
    out_ref[...] = jnp.maximum(acc, 0.0)


@jax.jit
def kernel(x, adj, W):
    out = pl.pallas_call(
        _fused_kernel,
        grid=(N // BM,),
        in_specs=[
            pl.BlockSpec((N, D_IN), lambda i: (0, 0)),
            pl.BlockSpec((D_IN, D_OUT), lambda i: (0, 0)),
            pl.BlockSpec((BM, N), lambda i: (i, 0)),
        ],
        out_specs=pl.BlockSpec((BM, D_OUT), lambda i: (i, 0)),
        out_shape=jax.ShapeDtypeStruct((N, D_OUT), jnp.float32),
        scratch_shapes=[pltpu.VMEM((N, D_OUT), jnp.float32)],
    )(x, W, adj)

    return (out, adj)
